# Initial kernel scaffold; baseline (speedup 1.0000x reference)
import jax
import jax.numpy as jnp
from jax.experimental import pallas as pl


def kernel(x, y, conv1_w, conv1_b, ln1_g, ln1_b, conv2_w, conv2_b, ln2_g, ln2_b, lin_w, lin_b):
    raise NotImplementedError("write your pallas kernel here")


# speedup vs baseline: 2.5434x; 2.5434x over previous
import functools

import jax
import jax.numpy as jnp
from jax import lax
from jax.experimental import pallas as pl
from jax.experimental.pallas import tpu as pltpu
from jax.experimental.pallas import tpu_sc as plsc

B, L, D = 8, 2048, 1024
NC, NS, LANE = 2, 16, 16
NW = NC * NS
RPW = B * L // NW
CHUNK = 16
NCHUNK = RPW // CHUNK
NVREG = L // LANE
BPC = B // NC


def _body(x_hbm, y_hbm, out_hbm, idx_hbm,
          yv, cum, marks, idxrow, idxv, rows_v, zrow, sem):
    c_id = lax.axis_index("c")
    s_id = lax.axis_index("s")
    wid = c_id * NS + s_id

    @pl.when(s_id < BPC)
    def _stage_a():
        b = c_id * BPC + s_id
        pltpu.sync_copy(y_hbm.at[b], yv)

        def cum_body(i, carry):
            v = yv[pl.ds(i * LANE, LANE)]
            li = (v + 0.5).astype(jnp.int32)
            s = plsc.cumsum(li) + carry
            cum[pl.ds(i * LANE, LANE)] = s
            return carry + jnp.sum(li)

        lax.fori_loop(0, NVREG, cum_body, jnp.int32(0))

        def zero_body(i, _):
            marks[pl.ds(i * LANE, LANE)] = jnp.zeros((LANE,), jnp.int32)
            return 0

        lax.fori_loop(0, NVREG, zero_body, 0)

        ones16 = jnp.ones((LANE,), jnp.int32)

        def mark_body(i, _):
            q = cum[pl.ds(i * LANE, LANE)]
            plsc.addupdate_scatter(marks, [q], ones16,
                                   mask=(q >= 0) & (q < L))
            return 0

        lax.fori_loop(0, NVREG, mark_body, 0)

        def idx_body(i, carry):
            m = marks[pl.ds(i * LANE, LANE)]
            s = plsc.cumsum(m) + carry
            idxrow[i] = s + b * L
            return carry + jnp.sum(m)

        lax.fori_loop(0, NVREG, idx_body, jnp.int32(0))
        pltpu.sync_copy(idxrow, idx_hbm.at[pl.ds(b * NVREG, NVREG)])

    plsc.subcore_barrier()

    base = wid * RPW
    b_w = wid // (L // RPW)
    lim = b_w * L + L - 1
    pltpu.sync_copy(idx_hbm.at[pl.ds(wid * NCHUNK, NCHUNK)], idxv)

    ninv = jnp.int32(0)
    for c in range(NCHUNK):
        iv = idxv[c]
        giv = jnp.minimum(iv, lim)
        pltpu.async_copy(x_hbm.at[giv], rows_v, sem).wait()
        pltpu.sync_copy(rows_v, out_hbm.at[pl.ds(base + c * CHUNK, CHUNK)])
        ninv = ninv + jnp.sum((iv > lim).astype(jnp.int32))

    @pl.when(ninv > 0)
    def _tail():
        def zb(i, _):
            zrow[pl.ds(i * LANE, LANE)] = jnp.zeros((LANE,), jnp.float32)
            return 0

        lax.fori_loop(0, D // LANE, zb, 0)

        def tz(p, _):
            pltpu.sync_copy(zrow, out_hbm.at[base + p])
            return 0

        lax.fori_loop(RPW - ninv, RPW, tz, 0)


@functools.partial(
    pl.kernel,
    out_type=(jax.ShapeDtypeStruct((B * L, D), jnp.float32),
              jax.ShapeDtypeStruct((B * NVREG, LANE), jnp.int32)),
    mesh=plsc.VectorSubcoreMesh(core_axis_name="c", subcore_axis_name="s"),
    compiler_params=pltpu.CompilerParams(needs_layout_passes=False),
    scratch_types=[
        pltpu.VMEM((L,), jnp.float32),
        pltpu.VMEM((L,), jnp.int32),
        pltpu.VMEM((L,), jnp.int32),
        pltpu.VMEM((NVREG, LANE), jnp.int32),
        pltpu.VMEM((NCHUNK, CHUNK), jnp.int32),
        pltpu.VMEM((CHUNK, D), jnp.float32),
        pltpu.VMEM((D,), jnp.float32),
        pltpu.SemaphoreType.DMA,
    ],
)
def _sc_expand(x_hbm, y_hbm, out_hbm, idx_hbm, *scratch):
    _body(x_hbm, y_hbm, out_hbm, idx_hbm, *scratch)


def kernel(x, y, conv1_w, conv1_b, ln1_g, ln1_b, conv2_w, conv2_b, ln2_g,
           ln2_b, lin_w, lin_b):
    out, _ = _sc_expand(x.reshape(B * L, D), y)
    return out.reshape(B, L, D)


# speedup vs baseline: 3.2663x; 1.2842x over previous
import functools

import jax
import jax.numpy as jnp
from jax import lax
from jax.experimental import pallas as pl
from jax.experimental.pallas import tpu as pltpu
from jax.experimental.pallas import tpu_sc as plsc

B, L, D = 8, 2048, 1024
NC, NS, LANE = 2, 16, 16
NW = NC * NS
RPW = B * L // NW
CHUNK = 16
NCHUNK = RPW // CHUNK
GRP = 2
NGRP = NCHUNK // GRP
NVREG = L // LANE
BPC = B // NC


def _body(x_hbm, y_hbm, out_hbm, idx_hbm,
          yv, marks, idxrow, idxv, rows_a, rows_b, zrow, sem_a, sem_b):
    c_id = lax.axis_index("c")
    s_id = lax.axis_index("s")
    wid = c_id * NS + s_id

    @pl.when(s_id < BPC)
    def _stage_a():
        b = c_id * BPC + s_id
        pltpu.sync_copy(y_hbm.at[b], yv)

        def zero_body(i, _):
            marks[pl.ds(i * LANE, LANE)] = jnp.zeros((LANE,), jnp.int32)
            return 0

        lax.fori_loop(0, NVREG, zero_body, 0)

        ones16 = jnp.ones((LANE,), jnp.int32)

        def cum_mark_body(i, carry):
            v = yv[pl.ds(i * LANE, LANE)]
            li = (v + 0.5).astype(jnp.int32)
            s = plsc.cumsum(li) + carry
            plsc.addupdate_scatter(marks, [s], ones16,
                                   mask=(s >= 0) & (s < L))
            return s[15]

        lax.fori_loop(0, NVREG, cum_mark_body, jnp.int32(0))

        def idx_body(i, carry):
            m = marks[pl.ds(i * LANE, LANE)]
            s = plsc.cumsum(m) + carry
            idxrow[i] = s + b * L
            return s[15]

        lax.fori_loop(0, NVREG, idx_body, jnp.int32(0))
        pltpu.sync_copy(idxrow, idx_hbm.at[pl.ds(b * NVREG, NVREG)])

    plsc.subcore_barrier()

    base = wid * RPW
    b_w = wid // (L // RPW)
    lim = b_w * L + L - 1
    pltpu.sync_copy(idx_hbm.at[pl.ds(wid * NCHUNK, NCHUNK)], idxv)

    ninv = jnp.int32(0)
    for c in range(NCHUNK):
        ninv = ninv + jnp.sum((idxv[c] > lim).astype(jnp.int32))

    bufs = (rows_a, rows_b)
    sems = (sem_a, sem_b)

    def fire(g):
        hs = []
        for k in range(GRP):
            giv = jnp.minimum(idxv[g * GRP + k], lim)
            hs.append(pltpu.async_copy(
                x_hbm.at[giv], bufs[g % 2].at[pl.ds(k * CHUNK, CHUNK)],
                sems[g % 2]))
        return hs

    handles = [fire(0), None]
    for g in range(NGRP):
        if g + 1 < NGRP:
            handles[(g + 1) % 2] = fire(g + 1)
        for h in handles[g % 2]:
            h.wait()
        pltpu.sync_copy(bufs[g % 2],
                        out_hbm.at[pl.ds(base + g * GRP * CHUNK, GRP * CHUNK)])

    @pl.when(ninv > 0)
    def _tail():
        def zb(i, _):
            zrow[pl.ds(i * LANE, LANE)] = jnp.zeros((LANE,), jnp.float32)
            return 0

        lax.fori_loop(0, D // LANE, zb, 0)

        def tz(p, _):
            pltpu.sync_copy(zrow, out_hbm.at[base + p])
            return 0

        lax.fori_loop(RPW - ninv, RPW, tz, 0)


@functools.partial(
    pl.kernel,
    out_type=(jax.ShapeDtypeStruct((B * L, D), jnp.float32),
              jax.ShapeDtypeStruct((B * NVREG, LANE), jnp.int32)),
    mesh=plsc.VectorSubcoreMesh(core_axis_name="c", subcore_axis_name="s"),
    compiler_params=pltpu.CompilerParams(needs_layout_passes=False),
    scratch_types=[
        pltpu.VMEM((L,), jnp.float32),
        pltpu.VMEM((L,), jnp.int32),
        pltpu.VMEM((NVREG, LANE), jnp.int32),
        pltpu.VMEM((NCHUNK, CHUNK), jnp.int32),
        pltpu.VMEM((GRP * CHUNK, D), jnp.float32),
        pltpu.VMEM((GRP * CHUNK, D), jnp.float32),
        pltpu.VMEM((D,), jnp.float32),
        pltpu.SemaphoreType.DMA,
        pltpu.SemaphoreType.DMA,
    ],
)
def _sc_expand(x_hbm, y_hbm, out_hbm, idx_hbm, *scratch):
    _body(x_hbm, y_hbm, out_hbm, idx_hbm, *scratch)


def kernel(x, y, conv1_w, conv1_b, ln1_g, ln1_b, conv2_w, conv2_b, ln2_g,
           ln2_b, lin_w, lin_b):
    out, _ = _sc_expand(x.reshape(B * L, D), y)
    return out.reshape(B, L, D)


# speedup vs baseline: 3.3932x; 1.0388x over previous
import functools

import jax
import jax.numpy as jnp
from jax import lax
from jax.experimental import pallas as pl
from jax.experimental.pallas import tpu as pltpu
from jax.experimental.pallas import tpu_sc as plsc

B, L, D = 8, 2048, 1024
NC, NS, LANE = 2, 16, 16
NW = NC * NS
RPW = B * L // NW
CHUNK = 16
NCHUNK = RPW // CHUNK
GRP = 2
NGRP = NCHUNK // GRP
NVREG = L // LANE
BPC = B // NC


def _body(x_hbm, y_hbm, out_hbm, idx_hbm,
          yv, marks, idxrow, idxv, rows_a, rows_b, zrow, sem_a, sem_b):
    c_id = lax.axis_index("c")
    s_id = lax.axis_index("s")
    wid = c_id * NS + s_id

    @pl.when(s_id < BPC)
    def _stage_a():
        b = c_id * BPC + s_id
        pltpu.sync_copy(y_hbm.at[b], yv)

        zeros16 = jnp.zeros((LANE,), jnp.int32)

        def zero_body(i, _):
            for k in range(8):
                marks[pl.ds((i * 8 + k) * LANE, LANE)] = zeros16
            return 0

        lax.fori_loop(0, NVREG // 8, zero_body, 0)

        ones16 = jnp.ones((LANE,), jnp.int32)

        def cum_mark_body(i, carry):
            ss = [plsc.cumsum(
                (yv[pl.ds((i * 4 + k) * LANE, LANE)] + 0.5).astype(jnp.int32))
                for k in range(4)]
            for k in range(4):
                s = ss[k] + carry
                plsc.addupdate_scatter(marks, [s], ones16,
                                       mask=(s >= 0) & (s < L))
                carry = s[15]
            return carry

        lax.fori_loop(0, NVREG // 4, cum_mark_body, jnp.int32(0))

        def idx_body(i, carry):
            ss = [plsc.cumsum(marks[pl.ds((i * 4 + k) * LANE, LANE)])
                  for k in range(4)]
            for k in range(4):
                s = ss[k] + carry
                idxrow[i * 4 + k] = s + b * L
                carry = s[15]
            return carry

        lax.fori_loop(0, NVREG // 4, idx_body, jnp.int32(0))
        pltpu.sync_copy(idxrow, idx_hbm.at[pl.ds(b * NVREG, NVREG)])

    plsc.subcore_barrier()

    base = wid * RPW
    b_w = wid // (L // RPW)
    lim = b_w * L + L - 1
    pltpu.sync_copy(idx_hbm.at[pl.ds(wid * NCHUNK, NCHUNK)], idxv)

    bufs = (rows_a, rows_b)
    sems = (sem_a, sem_b)
    ninv_parts = []

    def fire(g):
        hs = []
        for k in range(GRP):
            iv = idxv[g * GRP + k]
            ninv_parts.append(plsc.all_reduce_population_count(iv > lim)[0])
            hs.append(pltpu.async_copy(
                x_hbm.at[jnp.minimum(iv, lim)],
                bufs[g % 2].at[pl.ds(k * CHUNK, CHUNK)],
                sems[g % 2]))
        return hs

    handles = [fire(0), None]
    for g in range(NGRP):
        if g + 1 < NGRP:
            handles[(g + 1) % 2] = fire(g + 1)
        for h in handles[g % 2]:
            h.wait()
        pltpu.sync_copy(bufs[g % 2],
                        out_hbm.at[pl.ds(base + g * GRP * CHUNK, GRP * CHUNK)])

    ninv = jnp.int32(0)
    for p in ninv_parts:
        ninv = ninv + p

    @pl.when(ninv > 0)
    def _tail():
        def zb(i, _):
            zrow[pl.ds(i * LANE, LANE)] = jnp.zeros((LANE,), jnp.float32)
            return 0

        lax.fori_loop(0, D // LANE, zb, 0)

        def tz(p, _):
            pltpu.sync_copy(zrow, out_hbm.at[base + p])
            return 0

        lax.fori_loop(RPW - ninv, RPW, tz, 0)


@functools.partial(
    pl.kernel,
    out_type=(jax.ShapeDtypeStruct((B * L, D), jnp.float32),
              jax.ShapeDtypeStruct((B * NVREG, LANE), jnp.int32)),
    mesh=plsc.VectorSubcoreMesh(core_axis_name="c", subcore_axis_name="s"),
    compiler_params=pltpu.CompilerParams(needs_layout_passes=False),
    scratch_types=[
        pltpu.VMEM((L,), jnp.float32),
        pltpu.VMEM((L,), jnp.int32),
        pltpu.VMEM((NVREG, LANE), jnp.int32),
        pltpu.VMEM((NCHUNK, CHUNK), jnp.int32),
        pltpu.VMEM((GRP * CHUNK, D), jnp.float32),
        pltpu.VMEM((GRP * CHUNK, D), jnp.float32),
        pltpu.VMEM((D,), jnp.float32),
        pltpu.SemaphoreType.DMA,
        pltpu.SemaphoreType.DMA,
    ],
)
def _sc_expand(x_hbm, y_hbm, out_hbm, idx_hbm, *scratch):
    _body(x_hbm, y_hbm, out_hbm, idx_hbm, *scratch)


def kernel(x, y, conv1_w, conv1_b, ln1_g, ln1_b, conv2_w, conv2_b, ln2_g,
           ln2_b, lin_w, lin_b):
    out, _ = _sc_expand(x.reshape(B * L, D), y)
    return out.reshape(B, L, D)
